# baseline (speedup 1.0000x reference)
import functools

import jax
import jax.numpy as jnp
from jax import lax
from jax.experimental import pallas as pl
from jax.experimental.pallas import tpu as pltpu
from jax.experimental.pallas import tpu_sc as plsc

_B = 4096
_EMB = (2, 4, 8, 16, 64, 128)
_PITCH = (8, 8, 8, 16, 64, 128)
_SHIFT = (4, 4, 4, 3, 1, 0)
_MAXE = 128
_ODIM = 2
_EPS = 1e-5

_NC = 2
_NS = 16
_NW = _NC * _NS
_BPW = _B // _NW


def _sc_gather(uid, mid, tabs_u, tabs_m):
    mesh = plsc.VectorSubcoreMesh(
        core_axis_name="c", subcore_axis_name="s",
        num_cores=_NC, num_subcores=_NS)

    out_type = [jax.ShapeDtypeStruct((_B, 128), jnp.float32) for _ in range(12)]
    scratch_types = (
        [pltpu.VMEM((_BPW,), jnp.int32) for _ in range(14)]
        + [pltpu.VMEM((_BPW, 128), jnp.float32) for _ in range(6)]
        + [pltpu.SemaphoreType.DMA]
    )

    @functools.partial(pl.kernel, mesh=mesh, out_type=out_type,
                       scratch_types=scratch_types)
    def gather_kernel(uid_hbm, mid_hbm, *refs):
        utabs = refs[0:6]
        mtabs = refs[6:12]
        outs_u = refs[12:18]
        outs_m = refs[18:24]
        idx_u, idx_m = refs[24:26]
        idx_su = refs[26:32]
        idx_sm = refs[32:38]
        bufs = refs[38:44]
        sem = refs[44]

        wid = lax.axis_index("s") * _NC + lax.axis_index("c")
        base = wid * _BPW
        pltpu.sync_copy(uid_hbm.at[pl.ds(base, _BPW)], idx_u)
        pltpu.sync_copy(mid_hbm.at[pl.ds(base, _BPW)], idx_m)

        def shifted(src, dst, s):
            if s == 0:
                return src
            for c in range(_BPW // 16):
                sl = pl.ds(c * 16, 16)
                dst[sl] = lax.shift_right_logical(src[sl], s)
            return dst

        def wave(tabs, outs, idx, idx_shift):
            copies = []
            for j in range(6):
                ix = shifted(idx, idx_shift[j], _SHIFT[j])
                copies.append(pltpu.async_copy(tabs[j].at[ix], bufs[j], sem))
            for c in copies:
                c.wait()
            for j in range(6):
                pltpu.sync_copy(bufs[j], outs[j].at[pl.ds(base, _BPW)])

        wave(utabs, outs_u, idx_u, idx_su)
        wave(mtabs, outs_m, idx_m, idx_sm)

    return gather_kernel(uid, mid, *tabs_u, *tabs_m)


def _tc_body(su_ref, sm_ref, uid_ref, mid_ref,
             gu0, gu1, gu2, gu3, gu4, gu5,
             gm0, gm1, gm2, gm3, gm4, gm5,
             wu0, wu1, wu2, wu3, wu4, wu5,
             wm0, wm1, wm2, wm3, wm4, wm5,
             bu_ref, bm_ref,
             bnug_ref, bnub_ref, bnmg_ref, bnmb_ref,
             g1u_ref, b1u_ref, g1m_ref, b1m_ref,
             w1u_ref, w1m_ref, b1_ref,
             g2_ref, b2n_ref, w2_ref, b2_ref,
             out_ref):
    gus = (gu0, gu1, gu2, gu3, gu4, gu5)
    gms = (gm0, gm1, gm2, gm3, gm4, gm5)
    wus = (wu0, wu1, wu2, wu3, wu4, wu5)
    wms = (wm0, wm1, wm2, wm3, wm4, wm5)

    f32 = jnp.float32
    su = su_ref[...]
    sm = sm_ref[...]
    uid = uid_ref[...]
    mid = mid_ref[...]
    lane = lax.broadcasted_iota(jnp.int32, (_B, 128), 1)

    def unified(sizes, rid, gs, ws, bstack):
        acc = jnp.zeros((_B, _MAXE), dtype=f32)
        for j in range(6):
            p = _PITCH[j]
            nsub = 128 // p
            sel = (sizes == j).astype(f32)
            sub = lax.rem(rid, nsub) if nsub > 1 else None
            x = gs[j][...]
            if sub is not None:
                x = x * ((lane // p) == sub).astype(f32)
            acc = acc + jnp.dot(x * sel, ws[j][...], preferred_element_type=f32)
        onehot = (sizes == lax.broadcasted_iota(jnp.int32, (1, 8), 1)).astype(f32)
        return acc + jnp.dot(onehot, bstack, preferred_element_type=f32)

    def bn(x, g, b):
        m = jnp.mean(x, axis=0, keepdims=True)
        v = jnp.mean((x - m) ** 2, axis=0, keepdims=True)
        return (x - m) * lax.rsqrt(v + _EPS) * g + b

    uu = unified(su, uid, gus, wus, bu_ref[...])
    um = unified(sm, mid, gms, wms, bm_ref[...])
    vu = jnp.tanh(bn(uu, bnug_ref[...], bnub_ref[...]))
    vm = jnp.tanh(bn(um, bnmg_ref[...], bnmb_ref[...]))
    au = bn(vu, g1u_ref[...], b1u_ref[...])
    am = bn(vm, g1m_ref[...], b1m_ref[...])
    h = (jnp.dot(au, w1u_ref[...], preferred_element_type=f32)
         + jnp.dot(am, w1m_ref[...], preferred_element_type=f32)
         + b1_ref[...])
    h = jnp.tanh(bn(h, g2_ref[...], b2n_ref[...]))
    out_ref[...] = jnp.dot(h, w2_ref[...], preferred_element_type=f32) + b2_ref[...]


def _tab128(t, pitch):
    e = t.shape[1]
    if e < pitch:
        t = jnp.pad(t, ((0, 0), (0, pitch - e)))
    return t.reshape(-1, 128)


def _wstack(w, pitch):
    e = w.shape[1]
    wt = jnp.pad(w.T.astype(jnp.float32), ((0, pitch - e), (0, 0)))
    return jnp.tile(wt, (128 // pitch, 1))


def kernel(u_emb_sizes, m_emb_sizes, userID, movieID, movie_vec,
           emb_user, emb_movie, W_user_w, W_user_b, W_movie_w, W_movie_b,
           bn_user_g, bn_user_b, bn_movie_g, bn_movie_b,
           t_bn1_g, t_bn1_b, t_w1, t_b1, t_bn2_g, t_bn2_b, t_w2, t_b2):
    f32 = jnp.float32
    uid = userID.astype(jnp.int32)
    mid = movieID.astype(jnp.int32)
    su = u_emb_sizes.astype(jnp.int32).reshape(_B, 1)
    sm = m_emb_sizes.astype(jnp.int32).reshape(_B, 1)

    tabs_u = [_tab128(emb_user[j], _PITCH[j]) for j in range(6)]
    tabs_m = [_tab128(emb_movie[j], _PITCH[j]) for j in range(6)]
    gathered = _sc_gather(uid, mid, tabs_u, tabs_m)
    gu = gathered[0:6]
    gm = gathered[6:12]

    wus = [_wstack(W_user_w[j], _PITCH[j]) for j in range(6)]
    wms = [_wstack(W_movie_w[j], _PITCH[j]) for j in range(6)]
    bu = jnp.concatenate([jnp.stack(W_user_b), jnp.zeros((2, _MAXE), f32)], 0)
    bm = jnp.concatenate([jnp.stack(W_movie_b), jnp.zeros((2, _MAXE), f32)], 0)
    row = lambda x: x.reshape(1, -1).astype(f32)
    w1u = t_w1[:, :_MAXE].T.astype(f32)
    w1m = t_w1[:, _MAXE:].T.astype(f32)
    w2 = t_w2.T.astype(f32)

    args = ([su, sm, uid.reshape(_B, 1), mid.reshape(_B, 1)]
            + list(gu) + list(gm) + wus + wms
            + [bu, bm,
               row(bn_user_g), row(bn_user_b), row(bn_movie_g), row(bn_movie_b),
               row(t_bn1_g[:_MAXE]), row(t_bn1_b[:_MAXE]),
               row(t_bn1_g[_MAXE:]), row(t_bn1_b[_MAXE:]),
               w1u, w1m, row(t_b1),
               row(t_bn2_g), row(t_bn2_b), w2, row(t_b2)])

    out = pl.pallas_call(
        _tc_body,
        out_shape=jax.ShapeDtypeStruct((_B, _ODIM), f32),
    )(*args)
    return out


# speedup vs baseline: 1.0012x; 1.0012x over previous
import functools

import jax
import jax.numpy as jnp
from jax import lax
from jax.experimental import pallas as pl
from jax.experimental.pallas import tpu as pltpu
from jax.experimental.pallas import tpu_sc as plsc

_B = 4096
_EMB = (2, 4, 8, 16, 64, 128)
_PITCH = (8, 8, 8, 16, 64, 128)
_SHIFT = (4, 4, 4, 3, 1, 0)
_MAXE = 128
_ODIM = 2
_EPS = 1e-5

_NC = 2
_NS = 16
_NW = _NC * _NS
_BPW = _B // _NW


def _sc_gather(uid, mid, tabs_u, tabs_m):
    mesh = plsc.VectorSubcoreMesh(
        core_axis_name="c", subcore_axis_name="s",
        num_cores=_NC, num_subcores=_NS)

    out_type = [jax.ShapeDtypeStruct((_B, 128), jnp.float32) for _ in range(12)]
    scratch_types = (
        [pltpu.VMEM((_BPW,), jnp.int32) for _ in range(14)]
        + [pltpu.VMEM((_BPW, 128), jnp.float32) for _ in range(6)]
        + [pltpu.SemaphoreType.DMA]
    )

    @functools.partial(pl.kernel, mesh=mesh, out_type=out_type,
                       scratch_types=scratch_types,
                       compiler_params=pltpu.CompilerParams(
                           use_tc_tiling_on_sc=True))
    def gather_kernel(uid_hbm, mid_hbm, *refs):
        utabs = refs[0:6]
        mtabs = refs[6:12]
        outs_u = refs[12:18]
        outs_m = refs[18:24]
        idx_u, idx_m = refs[24:26]
        idx_su = refs[26:32]
        idx_sm = refs[32:38]
        bufs = refs[38:44]
        sem = refs[44]

        wid = lax.axis_index("s") * _NC + lax.axis_index("c")
        base = wid * _BPW
        pltpu.sync_copy(uid_hbm.at[pl.ds(base, _BPW)], idx_u)
        pltpu.sync_copy(mid_hbm.at[pl.ds(base, _BPW)], idx_m)

        def shifted(src, dst, s):
            if s == 0:
                return src
            for c in range(_BPW // 16):
                sl = pl.ds(c * 16, 16)
                dst[sl] = lax.shift_right_logical(src[sl], s)
            return dst

        def wave(tabs, outs, idx, idx_shift):
            copies = []
            for j in range(6):
                ix = shifted(idx, idx_shift[j], _SHIFT[j])
                copies.append(pltpu.async_copy(tabs[j].at[ix], bufs[j], sem))
            for c in copies:
                c.wait()
            for j in range(6):
                pltpu.sync_copy(bufs[j], outs[j].at[pl.ds(base, _BPW)])

        wave(utabs, outs_u, idx_u, idx_su)
        wave(mtabs, outs_m, idx_m, idx_sm)

    return gather_kernel(uid, mid, *tabs_u, *tabs_m)


def _tc_body(su_ref, sm_ref, uid_ref, mid_ref,
             gu0, gu1, gu2, gu3, gu4, gu5,
             gm0, gm1, gm2, gm3, gm4, gm5,
             wu0, wu1, wu2, wu3, wu4, wu5,
             wm0, wm1, wm2, wm3, wm4, wm5,
             bu_ref, bm_ref,
             bnug_ref, bnub_ref, bnmg_ref, bnmb_ref,
             g1u_ref, b1u_ref, g1m_ref, b1m_ref,
             w1u_ref, w1m_ref, b1_ref,
             g2_ref, b2n_ref, w2_ref, b2_ref,
             out_ref):
    gus = (gu0, gu1, gu2, gu3, gu4, gu5)
    gms = (gm0, gm1, gm2, gm3, gm4, gm5)
    wus = (wu0, wu1, wu2, wu3, wu4, wu5)
    wms = (wm0, wm1, wm2, wm3, wm4, wm5)

    f32 = jnp.float32
    su = su_ref[...]
    sm = sm_ref[...]
    uid = uid_ref[...]
    mid = mid_ref[...]
    lane = lax.broadcasted_iota(jnp.int32, (_B, 128), 1)

    def unified(sizes, rid, gs, ws, bstack):
        acc = jnp.zeros((_B, _MAXE), dtype=f32)
        for j in range(6):
            p = _PITCH[j]
            nsub = 128 // p
            sel = (sizes == j).astype(f32)
            sub = lax.rem(rid, nsub) if nsub > 1 else None
            x = gs[j][...]
            if sub is not None:
                x = x * ((lane // p) == sub).astype(f32)
            acc = acc + jnp.dot(x * sel, ws[j][...], preferred_element_type=f32)
        onehot = (sizes == lax.broadcasted_iota(jnp.int32, (1, 8), 1)).astype(f32)
        return acc + jnp.dot(onehot, bstack, preferred_element_type=f32)

    def bn(x, g, b):
        m = jnp.mean(x, axis=0, keepdims=True)
        v = jnp.mean((x - m) ** 2, axis=0, keepdims=True)
        return (x - m) * lax.rsqrt(v + _EPS) * g + b

    uu = unified(su, uid, gus, wus, bu_ref[...])
    um = unified(sm, mid, gms, wms, bm_ref[...])
    vu = jnp.tanh(bn(uu, bnug_ref[...], bnub_ref[...]))
    vm = jnp.tanh(bn(um, bnmg_ref[...], bnmb_ref[...]))
    au = bn(vu, g1u_ref[...], b1u_ref[...])
    am = bn(vm, g1m_ref[...], b1m_ref[...])
    h = (jnp.dot(au, w1u_ref[...], preferred_element_type=f32)
         + jnp.dot(am, w1m_ref[...], preferred_element_type=f32)
         + b1_ref[...])
    h = jnp.tanh(bn(h, g2_ref[...], b2n_ref[...]))
    out_ref[...] = jnp.dot(h, w2_ref[...], preferred_element_type=f32) + b2_ref[...]


def _tab128(t, pitch):
    e = t.shape[1]
    if e < pitch:
        t = jnp.pad(t, ((0, 0), (0, pitch - e)))
    return t.reshape(-1, 128)


def _wstack(w, pitch):
    e = w.shape[1]
    wt = jnp.pad(w.T.astype(jnp.float32), ((0, pitch - e), (0, 0)))
    return jnp.tile(wt, (128 // pitch, 1))


def kernel(u_emb_sizes, m_emb_sizes, userID, movieID, movie_vec,
           emb_user, emb_movie, W_user_w, W_user_b, W_movie_w, W_movie_b,
           bn_user_g, bn_user_b, bn_movie_g, bn_movie_b,
           t_bn1_g, t_bn1_b, t_w1, t_b1, t_bn2_g, t_bn2_b, t_w2, t_b2):
    f32 = jnp.float32
    uid = userID.astype(jnp.int32)
    mid = movieID.astype(jnp.int32)
    su = u_emb_sizes.astype(jnp.int32).reshape(_B, 1)
    sm = m_emb_sizes.astype(jnp.int32).reshape(_B, 1)

    tabs_u = [_tab128(emb_user[j], _PITCH[j]) for j in range(6)]
    tabs_m = [_tab128(emb_movie[j], _PITCH[j]) for j in range(6)]
    gathered = _sc_gather(uid, mid, tabs_u, tabs_m)
    gu = gathered[0:6]
    gm = gathered[6:12]

    wus = [_wstack(W_user_w[j], _PITCH[j]) for j in range(6)]
    wms = [_wstack(W_movie_w[j], _PITCH[j]) for j in range(6)]
    bu = jnp.concatenate([jnp.stack(W_user_b), jnp.zeros((2, _MAXE), f32)], 0)
    bm = jnp.concatenate([jnp.stack(W_movie_b), jnp.zeros((2, _MAXE), f32)], 0)
    row = lambda x: x.reshape(1, -1).astype(f32)
    w1u = t_w1[:, :_MAXE].T.astype(f32)
    w1m = t_w1[:, _MAXE:].T.astype(f32)
    w2 = t_w2.T.astype(f32)

    args = ([su, sm, uid.reshape(_B, 1), mid.reshape(_B, 1)]
            + list(gu) + list(gm) + wus + wms
            + [bu, bm,
               row(bn_user_g), row(bn_user_b), row(bn_movie_g), row(bn_movie_b),
               row(t_bn1_g[:_MAXE]), row(t_bn1_b[:_MAXE]),
               row(t_bn1_g[_MAXE:]), row(t_bn1_b[_MAXE:]),
               w1u, w1m, row(t_b1),
               row(t_bn2_g), row(t_bn2_b), w2, row(t_b2)])

    out = pl.pallas_call(
        _tc_body,
        out_shape=jax.ShapeDtypeStruct((_B, _ODIM), f32),
    )(*args)
    return out


# speedup vs baseline: 6.8264x; 6.8181x over previous
import functools

import jax
import jax.numpy as jnp
from jax import lax
from jax.experimental import pallas as pl
from jax.experimental.pallas import tpu as pltpu
from jax.experimental.pallas import tpu_sc as plsc

_B = 4096
_EMB = (2, 4, 8, 16, 64, 128)
_MAXE = 128
_ODIM = 2
_EPS = 1e-5

_NC = 2
_NS = 16
_NW = _NC * _NS
_BPW = _B // _NW
_CH = 8
_NCHUNK = _BPW // _CH


def _sc_gather(uid, mid, su, sm, tvs_u, tvs_m, tab128_u, tab128_m):
    mesh = plsc.VectorSubcoreMesh(
        core_axis_name="c", subcore_axis_name="s",
        num_cores=_NC, num_subcores=_NS)

    out_type = [jax.ShapeDtypeStruct((_B, 128), jnp.float32) for _ in range(2)]
    scratch_types = (
        [pltpu.VMEM((_BPW + 8,), jnp.int32) for _ in range(4)]
        + [pltpu.VMEM((_BPW, 128), jnp.float32)]
        + [pltpu.VMEM((64, 128), jnp.float32) for _ in range(_CH)]
        + [pltpu.VMEM((_BPW, 128), jnp.float32)]
        + [pltpu.SemaphoreType.DMA, pltpu.SemaphoreType.DMA]
    )

    @functools.partial(pl.kernel, mesh=mesh, out_type=out_type,
                       scratch_types=scratch_types,
                       compiler_params=pltpu.CompilerParams(
                           use_tc_tiling_on_sc=True,
                           disable_bounds_checks=True,
                           needs_layout_passes=False))
    def gather_kernel(uid_hbm, mid_hbm, su_hbm, sm_hbm, *refs):
        tvs = (refs[0:5], refs[5:10])
        t128 = (refs[10], refs[11])
        outs = (refs[12], refs[13])
        ids_v = (refs[14], refs[15])
        szs_v = (refs[16], refs[17])
        rows128 = refs[18]
        slots = refs[19:19 + _CH]
        xbuf = refs[19 + _CH]
        sem = refs[20 + _CH]
        sem2 = refs[21 + _CH]

        wid = lax.axis_index("s") * _NC + lax.axis_index("c")
        base = wid * _BPW
        pltpu.sync_copy(uid_hbm.at[pl.ds(base, _BPW)], ids_v[0].at[pl.ds(0, _BPW)])
        pltpu.sync_copy(mid_hbm.at[pl.ds(base, _BPW)], ids_v[1].at[pl.ds(0, _BPW)])
        pltpu.sync_copy(su_hbm.at[pl.ds(base, _BPW)], szs_v[0].at[pl.ds(0, _BPW)])
        pltpu.sync_copy(sm_hbm.at[pl.ds(base, _BPW)], szs_v[1].at[pl.ds(0, _BPW)])

        zero16 = jnp.zeros((16,), jnp.float32)
        lane16 = lax.iota(jnp.int32, 16)

        def wave(side):
            tv = tvs[side]
            idx_s = ids_v[side]
            sz_s = szs_v[side]
            pltpu.async_copy(
                t128[side].at[idx_s.at[pl.ds(0, _BPW)]], rows128, sem2).wait()

            def chunk_body(chunk, carry):
                ivec = idx_s[pl.ds(chunk * _CH, 16)]
                svec = sz_s[pl.ds(chunk * _CH, 16)]
                for kk in range(_CH):
                    j = svec[kk]
                    cb = pl.multiple_of(lax.div(ivec[kk], 128) * 128, 128)

                    def fire(jj, kk=kk, cb=cb):
                        def f():
                            e = _EMB[jj]
                            pltpu.async_copy(
                                tv[jj].at[:, pl.ds(cb, 128)],
                                slots[kk].at[pl.ds(0, e)], sem)
                        return f
                    lax.switch(j, [fire(0), fire(1), fire(2), fire(3),
                                   fire(4), lambda: None])
                for kk in range(_CH):
                    j = svec[kk]

                    def drain(jj, kk=kk):
                        def f():
                            e = _EMB[jj]
                            pltpu.make_async_copy(
                                tv[jj].at[:, pl.ds(0, 128)],
                                slots[kk].at[pl.ds(0, e)], sem).wait()
                        return f
                    lax.switch(j, [drain(0), drain(1), drain(2), drain(3),
                                   drain(4), lambda: None])
                for kk in range(_CH):
                    j = svec[kk]
                    col = lax.rem(ivec[kk], 128)
                    i = chunk * _CH + kk

                    def extract(jj, kk=kk, col=col, i=i):
                        def f():
                            e = _EMB[jj]
                            cvec = jnp.broadcast_to(col, (16,))
                            for c in range(8):
                                if c * 16 < e:
                                    v = plsc.load_gather(
                                        slots[kk],
                                        [lane16 + c * 16, cvec],
                                        mask=(lane16 + c * 16) < e)
                                    v = jnp.where((lane16 + c * 16) < e,
                                                  v, zero16)
                                else:
                                    v = zero16
                                xbuf[i, pl.ds(c * 16, 16)] = v
                        return f

                    def extract128(i=i):
                        for c in range(8):
                            xbuf[i, pl.ds(c * 16, 16)] = \
                                rows128[i, pl.ds(c * 16, 16)]
                    lax.switch(j, [extract(0), extract(1), extract(2),
                                   extract(3), extract(4), extract128])
                return carry

            lax.fori_loop(0, _NCHUNK, chunk_body, 0)
            pltpu.sync_copy(xbuf, outs[side].at[pl.ds(base, _BPW)])

        wave(0)
        wave(1)

    return gather_kernel(uid, mid, su, sm, *tvs_u, *tvs_m, tab128_u, tab128_m)


def _tc_body(su_ref, sm_ref,
             xu_ref, xm_ref,
             wu0, wu1, wu2, wu3, wu4, wu5,
             wm0, wm1, wm2, wm3, wm4, wm5,
             bu_ref, bm_ref,
             bnug_ref, bnub_ref, bnmg_ref, bnmb_ref,
             g1u_ref, b1u_ref, g1m_ref, b1m_ref,
             w1u_ref, w1m_ref, b1_ref,
             g2_ref, b2n_ref, w2_ref, b2_ref,
             out_ref):
    wus = (wu0, wu1, wu2, wu3, wu4, wu5)
    wms = (wm0, wm1, wm2, wm3, wm4, wm5)

    f32 = jnp.float32
    su = su_ref[...]
    sm = sm_ref[...]

    def unified(sizes, x, ws, bstack):
        acc = jnp.zeros((_B, _MAXE), dtype=f32)
        for j in range(6):
            sel = (sizes == j).astype(f32)
            acc = acc + jnp.dot(x * sel, ws[j][...], preferred_element_type=f32)
        onehot = (sizes == lax.broadcasted_iota(jnp.int32, (1, 8), 1)).astype(f32)
        return acc + jnp.dot(onehot, bstack, preferred_element_type=f32)

    def bn(x, g, b):
        m = jnp.mean(x, axis=0, keepdims=True)
        v = jnp.mean((x - m) ** 2, axis=0, keepdims=True)
        return (x - m) * lax.rsqrt(v + _EPS) * g + b

    uu = unified(su, xu_ref[...], wus, bu_ref[...])
    um = unified(sm, xm_ref[...], wms, bm_ref[...])
    vu = jnp.tanh(bn(uu, bnug_ref[...], bnub_ref[...]))
    vm = jnp.tanh(bn(um, bnmg_ref[...], bnmb_ref[...]))
    au = bn(vu, g1u_ref[...], b1u_ref[...])
    am = bn(vm, g1m_ref[...], b1m_ref[...])
    h = (jnp.dot(au, w1u_ref[...], preferred_element_type=f32)
         + jnp.dot(am, w1m_ref[...], preferred_element_type=f32)
         + b1_ref[...])
    h = jnp.tanh(bn(h, g2_ref[...], b2n_ref[...]))
    out_ref[...] = jnp.dot(h, w2_ref[...], preferred_element_type=f32) + b2_ref[...]


def kernel(u_emb_sizes, m_emb_sizes, userID, movieID, movie_vec,
           emb_user, emb_movie, W_user_w, W_user_b, W_movie_w, W_movie_b,
           bn_user_g, bn_user_b, bn_movie_g, bn_movie_b,
           t_bn1_g, t_bn1_b, t_w1, t_b1, t_bn2_g, t_bn2_b, t_w2, t_b2):
    f32 = jnp.float32
    uid = userID.astype(jnp.int32)
    mid = movieID.astype(jnp.int32)
    su = u_emb_sizes.astype(jnp.int32)
    sm = m_emb_sizes.astype(jnp.int32)

    tvs_u = [emb_user[j].T for j in range(5)]
    tvs_m = [emb_movie[j].T for j in range(5)]
    xu, xm = _sc_gather(uid, mid, su, sm, tvs_u, tvs_m,
                        emb_user[5], emb_movie[5])

    wpad = lambda w: jnp.pad(w.T.astype(f32), ((0, _MAXE - w.shape[1]), (0, 0)))
    wus = [wpad(W_user_w[j]) for j in range(6)]
    wms = [wpad(W_movie_w[j]) for j in range(6)]
    bu = jnp.concatenate([jnp.stack(W_user_b), jnp.zeros((2, _MAXE), f32)], 0)
    bm = jnp.concatenate([jnp.stack(W_movie_b), jnp.zeros((2, _MAXE), f32)], 0)
    row = lambda x: x.reshape(1, -1).astype(f32)
    w1u = t_w1[:, :_MAXE].T.astype(f32)
    w1m = t_w1[:, _MAXE:].T.astype(f32)
    w2 = t_w2.T.astype(f32)

    args = ([su.reshape(_B, 1), sm.reshape(_B, 1), xu, xm]
            + wus + wms
            + [bu, bm,
               row(bn_user_g), row(bn_user_b), row(bn_movie_g), row(bn_movie_b),
               row(t_bn1_g[:_MAXE]), row(t_bn1_b[:_MAXE]),
               row(t_bn1_g[_MAXE:]), row(t_bn1_b[_MAXE:]),
               w1u, w1m, row(t_b1),
               row(t_bn2_g), row(t_bn2_b), w2, row(t_b2)])

    out = pl.pallas_call(
        _tc_body,
        out_shape=jax.ShapeDtypeStruct((_B, _ODIM), f32),
    )(*args)
    return out


# speedup vs baseline: 6.8567x; 1.0044x over previous
import functools

import jax
import jax.numpy as jnp
from jax import lax
from jax.experimental import pallas as pl
from jax.experimental.pallas import tpu as pltpu
from jax.experimental.pallas import tpu_sc as plsc

_B = 4096
_EMB = (2, 4, 8, 16, 64, 128)
_MAXE = 128
_ODIM = 2
_EPS = 1e-5

_NC = 2
_NS = 16
_NW = _NC * _NS
_BPW = _B // _NW
_CH = 8
_NCHUNK = _BPW // _CH


def _sc_gather(uid, mid, su, sm, tvs_u, tvs_m, tab128_u, tab128_m):
    mesh = plsc.VectorSubcoreMesh(
        core_axis_name="c", subcore_axis_name="s",
        num_cores=_NC, num_subcores=_NS)

    out_type = [jax.ShapeDtypeStruct((_B, 128), jnp.float32) for _ in range(2)]
    scratch_types = (
        [pltpu.VMEM((_BPW + 8,), jnp.int32) for _ in range(4)]
        + [pltpu.VMEM((64, 128), jnp.float32) for _ in range(_CH)]
        + [pltpu.VMEM((_BPW, 128), jnp.float32)]
        + [pltpu.SemaphoreType.DMA, pltpu.SemaphoreType.DMA]
    )

    @functools.partial(pl.kernel, mesh=mesh, out_type=out_type,
                       scratch_types=scratch_types,
                       compiler_params=pltpu.CompilerParams(
                           use_tc_tiling_on_sc=True,
                           disable_bounds_checks=True,
                           needs_layout_passes=False))
    def gather_kernel(uid_hbm, mid_hbm, su_hbm, sm_hbm, *refs):
        tvs = (refs[0:5], refs[5:10])
        t128 = (refs[10], refs[11])
        outs = (refs[12], refs[13])
        ids_v = (refs[14], refs[15])
        szs_v = (refs[16], refs[17])
        slots = refs[18:18 + _CH]
        xbuf = refs[18 + _CH]
        sem = refs[19 + _CH]
        sem2 = refs[20 + _CH]

        wid = lax.axis_index("s") * _NC + lax.axis_index("c")
        base = wid * _BPW
        pltpu.sync_copy(uid_hbm.at[pl.ds(base, _BPW)], ids_v[0].at[pl.ds(0, _BPW)])
        pltpu.sync_copy(mid_hbm.at[pl.ds(base, _BPW)], ids_v[1].at[pl.ds(0, _BPW)])
        pltpu.sync_copy(su_hbm.at[pl.ds(base, _BPW)], szs_v[0].at[pl.ds(0, _BPW)])
        pltpu.sync_copy(sm_hbm.at[pl.ds(base, _BPW)], szs_v[1].at[pl.ds(0, _BPW)])

        zero16 = jnp.zeros((16,), jnp.float32)
        lane16 = lax.iota(jnp.int32, 16)

        def wave(side):
            tv = tvs[side]
            idx_s = ids_v[side]
            sz_s = szs_v[side]
            pltpu.async_copy(
                t128[side].at[idx_s.at[pl.ds(0, _BPW)]], xbuf, sem2).wait()

            def chunk_body(chunk, carry):
                ivec = idx_s[pl.ds(chunk * _CH, 16)]
                svec = sz_s[pl.ds(chunk * _CH, 16)]
                for kk in range(_CH):
                    j = svec[kk]
                    cb = pl.multiple_of(lax.div(ivec[kk], 128) * 128, 128)

                    def fire(jj, kk=kk, cb=cb):
                        def f():
                            e = _EMB[jj]
                            pltpu.async_copy(
                                tv[jj].at[:, pl.ds(cb, 128)],
                                slots[kk].at[pl.ds(0, e)], sem)
                        return f
                    lax.switch(j, [fire(0), fire(1), fire(2), fire(3),
                                   fire(4), lambda: None])
                for kk in range(_CH):
                    j = svec[kk]

                    def drain(jj, kk=kk):
                        def f():
                            e = _EMB[jj]
                            pltpu.make_async_copy(
                                tv[jj].at[:, pl.ds(0, 128)],
                                slots[kk].at[pl.ds(0, e)], sem).wait()
                        return f
                    lax.switch(j, [drain(0), drain(1), drain(2), drain(3),
                                   drain(4), lambda: None])
                for kk in range(_CH):
                    j = svec[kk]
                    col = lax.rem(ivec[kk], 128)
                    i = chunk * _CH + kk

                    def extract(jj, kk=kk, col=col, i=i):
                        def f():
                            e = _EMB[jj]
                            cvec = jnp.broadcast_to(col, (16,))
                            for c in range(8):
                                if c * 16 < e:
                                    v = plsc.load_gather(
                                        slots[kk],
                                        [lane16 + c * 16, cvec],
                                        mask=(lane16 + c * 16) < e)
                                    v = jnp.where((lane16 + c * 16) < e,
                                                  v, zero16)
                                else:
                                    v = zero16
                                xbuf[i, pl.ds(c * 16, 16)] = v
                        return f

                    lax.switch(j, [extract(0), extract(1), extract(2),
                                   extract(3), extract(4), lambda: None])
                return carry

            lax.fori_loop(0, _NCHUNK, chunk_body, 0)
            pltpu.sync_copy(xbuf, outs[side].at[pl.ds(base, _BPW)])

        wave(0)
        wave(1)

    return gather_kernel(uid, mid, su, sm, *tvs_u, *tvs_m, tab128_u, tab128_m)


def _tc_body(su_ref, sm_ref,
             xu_ref, xm_ref,
             wu0, wu1, wu2, wu3, wu4, wu5,
             wm0, wm1, wm2, wm3, wm4, wm5,
             bu_ref, bm_ref,
             bnug_ref, bnub_ref, bnmg_ref, bnmb_ref,
             g1u_ref, b1u_ref, g1m_ref, b1m_ref,
             w1u_ref, w1m_ref, b1_ref,
             g2_ref, b2n_ref, w2_ref, b2_ref,
             out_ref):
    wus = (wu0, wu1, wu2, wu3, wu4, wu5)
    wms = (wm0, wm1, wm2, wm3, wm4, wm5)

    f32 = jnp.float32
    su = su_ref[...]
    sm = sm_ref[...]

    def unified(sizes, x, ws, bstack):
        acc = jnp.zeros((_B, _MAXE), dtype=f32)
        for j in range(6):
            sel = (sizes == j).astype(f32)
            acc = acc + jnp.dot(x * sel, ws[j][...], preferred_element_type=f32)
        onehot = (sizes == lax.broadcasted_iota(jnp.int32, (1, 8), 1)).astype(f32)
        return acc + jnp.dot(onehot, bstack, preferred_element_type=f32)

    def bn(x, g, b):
        m = jnp.mean(x, axis=0, keepdims=True)
        v = jnp.mean((x - m) ** 2, axis=0, keepdims=True)
        return (x - m) * lax.rsqrt(v + _EPS) * g + b

    uu = unified(su, xu_ref[...], wus, bu_ref[...])
    um = unified(sm, xm_ref[...], wms, bm_ref[...])
    vu = jnp.tanh(bn(uu, bnug_ref[...], bnub_ref[...]))
    vm = jnp.tanh(bn(um, bnmg_ref[...], bnmb_ref[...]))
    au = bn(vu, g1u_ref[...], b1u_ref[...])
    am = bn(vm, g1m_ref[...], b1m_ref[...])
    h = (jnp.dot(au, w1u_ref[...], preferred_element_type=f32)
         + jnp.dot(am, w1m_ref[...], preferred_element_type=f32)
         + b1_ref[...])
    h = jnp.tanh(bn(h, g2_ref[...], b2n_ref[...]))
    out_ref[...] = jnp.dot(h, w2_ref[...], preferred_element_type=f32) + b2_ref[...]


def kernel(u_emb_sizes, m_emb_sizes, userID, movieID, movie_vec,
           emb_user, emb_movie, W_user_w, W_user_b, W_movie_w, W_movie_b,
           bn_user_g, bn_user_b, bn_movie_g, bn_movie_b,
           t_bn1_g, t_bn1_b, t_w1, t_b1, t_bn2_g, t_bn2_b, t_w2, t_b2):
    f32 = jnp.float32
    uid = userID.astype(jnp.int32)
    mid = movieID.astype(jnp.int32)
    su = u_emb_sizes.astype(jnp.int32)
    sm = m_emb_sizes.astype(jnp.int32)

    tvs_u = [emb_user[j].T for j in range(5)]
    tvs_m = [emb_movie[j].T for j in range(5)]
    xu, xm = _sc_gather(uid, mid, su, sm, tvs_u, tvs_m,
                        emb_user[5], emb_movie[5])

    wpad = lambda w: jnp.pad(w.T.astype(f32), ((0, _MAXE - w.shape[1]), (0, 0)))
    wus = [wpad(W_user_w[j]) for j in range(6)]
    wms = [wpad(W_movie_w[j]) for j in range(6)]
    bu = jnp.concatenate([jnp.stack(W_user_b), jnp.zeros((2, _MAXE), f32)], 0)
    bm = jnp.concatenate([jnp.stack(W_movie_b), jnp.zeros((2, _MAXE), f32)], 0)
    row = lambda x: x.reshape(1, -1).astype(f32)
    w1u = t_w1[:, :_MAXE].T.astype(f32)
    w1m = t_w1[:, _MAXE:].T.astype(f32)
    w2 = t_w2.T.astype(f32)

    args = ([su.reshape(_B, 1), sm.reshape(_B, 1), xu, xm]
            + wus + wms
            + [bu, bm,
               row(bn_user_g), row(bn_user_b), row(bn_movie_g), row(bn_movie_b),
               row(t_bn1_g[:_MAXE]), row(t_bn1_b[:_MAXE]),
               row(t_bn1_g[_MAXE:]), row(t_bn1_b[_MAXE:]),
               w1u, w1m, row(t_b1),
               row(t_bn2_g), row(t_bn2_b), w2, row(t_b2)])

    out = pl.pallas_call(
        _tc_body,
        out_shape=jax.ShapeDtypeStruct((_B, _ODIM), f32),
    )(*args)
    return out


# speedup vs baseline: 7.1177x; 1.0381x over previous
import functools

import jax
import jax.numpy as jnp
from jax import lax
from jax.experimental import pallas as pl
from jax.experimental.pallas import tpu as pltpu
from jax.experimental.pallas import tpu_sc as plsc

_B = 4096
_EMB = (2, 4, 8, 16, 64, 128)
_MAXE = 128
_ODIM = 2
_EPS = 1e-5

_NC = 2
_NS = 16
_NW = _NC * _NS
_BPW = _B // _NW
_CH = 8
_NCHUNK = _BPW // _CH


def _sc_gather(uid, mid, su, sm, tvs_u, tvs_m, tab128_u, tab128_m):
    mesh = plsc.VectorSubcoreMesh(
        core_axis_name="c", subcore_axis_name="s",
        num_cores=_NC, num_subcores=_NS)

    out_type = [jax.ShapeDtypeStruct((_B, 128), jnp.float32) for _ in range(2)]
    scratch_types = (
        [pltpu.VMEM((_BPW + 8,), jnp.int32) for _ in range(4)]
        + [pltpu.VMEM((64, 128), jnp.float32) for _ in range(_CH)]
        + [pltpu.VMEM((_BPW, 128), jnp.float32)]
        + [pltpu.SemaphoreType.DMA for _ in range(_CH + 1)]
    )

    @functools.partial(pl.kernel, mesh=mesh, out_type=out_type,
                       scratch_types=scratch_types,
                       compiler_params=pltpu.CompilerParams(
                           use_tc_tiling_on_sc=True,
                           disable_bounds_checks=True,
                           needs_layout_passes=False))
    def gather_kernel(uid_hbm, mid_hbm, su_hbm, sm_hbm, *refs):
        tvs = (refs[0:5], refs[5:10])
        t128 = (refs[10], refs[11])
        outs = (refs[12], refs[13])
        ids_v = (refs[14], refs[15])
        szs_v = (refs[16], refs[17])
        slots = refs[18:18 + _CH]
        xbuf = refs[18 + _CH]
        sems = refs[19 + _CH:19 + 2 * _CH]
        sem2 = refs[19 + 2 * _CH]

        wid = lax.axis_index("s") * _NC + lax.axis_index("c")
        base = wid * _BPW
        pltpu.sync_copy(uid_hbm.at[pl.ds(base, _BPW)], ids_v[0].at[pl.ds(0, _BPW)])
        pltpu.sync_copy(mid_hbm.at[pl.ds(base, _BPW)], ids_v[1].at[pl.ds(0, _BPW)])
        pltpu.sync_copy(su_hbm.at[pl.ds(base, _BPW)], szs_v[0].at[pl.ds(0, _BPW)])
        pltpu.sync_copy(sm_hbm.at[pl.ds(base, _BPW)], szs_v[1].at[pl.ds(0, _BPW)])

        zero16 = jnp.zeros((16,), jnp.float32)
        lane16 = lax.iota(jnp.int32, 16)

        def wave(side):
            tv = tvs[side]
            idx_s = ids_v[side]
            sz_s = szs_v[side]
            pltpu.async_copy(
                t128[side].at[idx_s.at[pl.ds(0, _BPW)]], xbuf, sem2).wait()

            def chunk_body(chunk, carry):
                ivec = idx_s[pl.ds(chunk * _CH, 16)]
                svec = sz_s[pl.ds(chunk * _CH, 16)]
                for kk in range(_CH):
                    j = svec[kk]
                    cb = pl.multiple_of(lax.div(ivec[kk], 128) * 128, 128)

                    def fire(jj, kk=kk, cb=cb):
                        def f():
                            e = _EMB[jj]
                            pltpu.async_copy(
                                tv[jj].at[:, pl.ds(cb, 128)],
                                slots[kk].at[pl.ds(0, e)], sems[kk])
                        return f
                    lax.switch(j, [fire(0), fire(1), fire(2), fire(3),
                                   fire(4), lambda: None])
                for kk in range(_CH):
                    j = svec[kk]

                    def drain(jj, kk=kk):
                        def f():
                            e = _EMB[jj]
                            pltpu.make_async_copy(
                                tv[jj].at[:, pl.ds(0, 128)],
                                slots[kk].at[pl.ds(0, e)], sems[kk]).wait()
                        return f
                    lax.switch(j, [drain(0), drain(1), drain(2), drain(3),
                                   drain(4), lambda: None])
                    col = lax.rem(ivec[kk], 128)
                    i = chunk * _CH + kk

                    def extract(jj, kk=kk, col=col, i=i):
                        def f():
                            e = _EMB[jj]
                            cvec = jnp.broadcast_to(col, (16,))
                            for c in range(8):
                                if c * 16 < e:
                                    v = plsc.load_gather(
                                        slots[kk],
                                        [lane16 + c * 16, cvec],
                                        mask=(lane16 + c * 16) < e)
                                    v = jnp.where((lane16 + c * 16) < e,
                                                  v, zero16)
                                else:
                                    v = zero16
                                xbuf[i, pl.ds(c * 16, 16)] = v
                        return f

                    lax.switch(j, [extract(0), extract(1), extract(2),
                                   extract(3), extract(4), lambda: None])
                return carry

            lax.fori_loop(0, _NCHUNK, chunk_body, 0)
            pltpu.sync_copy(xbuf, outs[side].at[pl.ds(base, _BPW)])

        wave(0)
        wave(1)

    return gather_kernel(uid, mid, su, sm, *tvs_u, *tvs_m, tab128_u, tab128_m)


def _tc_body(su_ref, sm_ref,
             xu_ref, xm_ref,
             wu0, wu1, wu2, wu3, wu4, wu5,
             wm0, wm1, wm2, wm3, wm4, wm5,
             bu_ref, bm_ref,
             bnug_ref, bnub_ref, bnmg_ref, bnmb_ref,
             g1u_ref, b1u_ref, g1m_ref, b1m_ref,
             w1u_ref, w1m_ref, b1_ref,
             g2_ref, b2n_ref, w2_ref, b2_ref,
             out_ref):
    wus = (wu0, wu1, wu2, wu3, wu4, wu5)
    wms = (wm0, wm1, wm2, wm3, wm4, wm5)

    f32 = jnp.float32
    su = su_ref[...]
    sm = sm_ref[...]

    def unified(sizes, x, ws, bstack):
        acc = jnp.zeros((_B, _MAXE), dtype=f32)
        for j in range(6):
            sel = (sizes == j).astype(f32)
            acc = acc + jnp.dot(x * sel, ws[j][...], preferred_element_type=f32)
        onehot = (sizes == lax.broadcasted_iota(jnp.int32, (1, 8), 1)).astype(f32)
        return acc + jnp.dot(onehot, bstack, preferred_element_type=f32)

    def bn(x, g, b):
        m = jnp.mean(x, axis=0, keepdims=True)
        v = jnp.mean((x - m) ** 2, axis=0, keepdims=True)
        return (x - m) * lax.rsqrt(v + _EPS) * g + b

    uu = unified(su, xu_ref[...], wus, bu_ref[...])
    um = unified(sm, xm_ref[...], wms, bm_ref[...])
    vu = jnp.tanh(bn(uu, bnug_ref[...], bnub_ref[...]))
    vm = jnp.tanh(bn(um, bnmg_ref[...], bnmb_ref[...]))
    au = bn(vu, g1u_ref[...], b1u_ref[...])
    am = bn(vm, g1m_ref[...], b1m_ref[...])
    h = (jnp.dot(au, w1u_ref[...], preferred_element_type=f32)
         + jnp.dot(am, w1m_ref[...], preferred_element_type=f32)
         + b1_ref[...])
    h = jnp.tanh(bn(h, g2_ref[...], b2n_ref[...]))
    out_ref[...] = jnp.dot(h, w2_ref[...], preferred_element_type=f32) + b2_ref[...]


def kernel(u_emb_sizes, m_emb_sizes, userID, movieID, movie_vec,
           emb_user, emb_movie, W_user_w, W_user_b, W_movie_w, W_movie_b,
           bn_user_g, bn_user_b, bn_movie_g, bn_movie_b,
           t_bn1_g, t_bn1_b, t_w1, t_b1, t_bn2_g, t_bn2_b, t_w2, t_b2):
    f32 = jnp.float32
    uid = userID.astype(jnp.int32)
    mid = movieID.astype(jnp.int32)
    su = u_emb_sizes.astype(jnp.int32)
    sm = m_emb_sizes.astype(jnp.int32)

    tvs_u = [emb_user[j].T for j in range(5)]
    tvs_m = [emb_movie[j].T for j in range(5)]
    xu, xm = _sc_gather(uid, mid, su, sm, tvs_u, tvs_m,
                        emb_user[5], emb_movie[5])

    wpad = lambda w: jnp.pad(w.T.astype(f32), ((0, _MAXE - w.shape[1]), (0, 0)))
    wus = [wpad(W_user_w[j]) for j in range(6)]
    wms = [wpad(W_movie_w[j]) for j in range(6)]
    bu = jnp.concatenate([jnp.stack(W_user_b), jnp.zeros((2, _MAXE), f32)], 0)
    bm = jnp.concatenate([jnp.stack(W_movie_b), jnp.zeros((2, _MAXE), f32)], 0)
    row = lambda x: x.reshape(1, -1).astype(f32)
    w1u = t_w1[:, :_MAXE].T.astype(f32)
    w1m = t_w1[:, _MAXE:].T.astype(f32)
    w2 = t_w2.T.astype(f32)

    args = ([su.reshape(_B, 1), sm.reshape(_B, 1), xu, xm]
            + wus + wms
            + [bu, bm,
               row(bn_user_g), row(bn_user_b), row(bn_movie_g), row(bn_movie_b),
               row(t_bn1_g[:_MAXE]), row(t_bn1_b[:_MAXE]),
               row(t_bn1_g[_MAXE:]), row(t_bn1_b[_MAXE:]),
               w1u, w1m, row(t_b1),
               row(t_bn2_g), row(t_bn2_b), w2, row(t_b2)])

    out = pl.pallas_call(
        _tc_body,
        out_shape=jax.ShapeDtypeStruct((_B, _ODIM), f32),
    )(*args)
    return out
